# baseline (speedup 1.0000x reference)
import functools

import jax
import jax.numpy as jnp
from jax import lax
from jax.experimental import pallas as pl
from jax.experimental.pallas import tpu as pltpu
from jax.experimental.pallas import tpu_sc as plsc

N = 10000
E = 320000
D = 128

NC = 2
NS = 16
NW = NC * NS
NP = 10240
SLAB = NP // NS

CW = 128
PAIR = 160
W0 = 128
W1 = PAIR - W0
EPAD = NS * PAIR * CW
NROWS = EPAD // CW
BI = 8


def _make_agg(with_cnt: bool):
    mesh = plsc.VectorSubcoreMesh(core_axis_name="c", subcore_axis_name="s")

    out_type = [jax.ShapeDtypeStruct((NC * NP, D), jnp.float32)]
    scratch = [
        pltpu.VMEM((BI, CW), jnp.int32),
        pltpu.VMEM((BI, CW), jnp.int32),
        pltpu.VMEM((CW, D), jnp.float32),
        pltpu.VMEM((CW, D), jnp.float32),
        pltpu.SemaphoreType.DMA,
        pltpu.SemaphoreType.DMA,
        pltpu.SemaphoreType.DMA,
        pltpu.SemaphoreType.DMA,
        pltpu.VMEM_SHARED((NP, D), jnp.float32),
    ]
    if with_cnt:
        out_type.append(jax.ShapeDtypeStruct((NC * NP,), jnp.float32))
        scratch += [
            pltpu.VMEM((CW,), jnp.float32),
            pltpu.VMEM_SHARED((NP,), jnp.float32),
        ]

    def body(*refs):
        if with_cnt:
            (h_hbm, src_hbm, dst_hbm, znd_hbm, zn_hbm, out_hbm, cnt_hbm,
             src_v, dst_v, bufA, bufB, gsA, ssA, gsB, ssB, agg_s,
             ones_v, cnt_s) = refs
        else:
            (h_hbm, src_hbm, dst_hbm, znd_hbm, out_hbm,
             src_v, dst_v, bufA, bufB, gsA, ssA, gsB, ssB, agg_s) = refs

        cid = lax.axis_index("c")
        sid = lax.axis_index("s")
        wid = sid * NC + cid

        pltpu.sync_copy(znd_hbm.at[pl.ds(sid * SLAB, SLAB)],
                        agg_s.at[pl.ds(sid * SLAB, SLAB)])
        if with_cnt:
            pltpu.sync_copy(zn_hbm.at[pl.ds(sid * SLAB, SLAB)],
                            cnt_s.at[pl.ds(sid * SLAB, SLAB)])
            for i in range(CW // 16):
                ones_v[pl.ds(i * 16, 16)] = jnp.full((16,), 1.0, jnp.float32)
        plsc.subcore_barrier()

        def gather(j, buf, sem):
            pltpu.async_copy(h_hbm.at[src_v.at[j]], buf, sem)

        def gwait(buf, sem):
            pltpu.make_async_copy(h_hbm.at[src_v.at[0]], buf, sem).wait()

        def scat(j, buf, sem):
            pltpu.async_copy(buf, agg_s.at[dst_v.at[j]], sem, add=True)
            if with_cnt:
                pltpu.async_copy(ones_v, cnt_s.at[dst_v.at[j]], sem, add=True)

        def swait(buf, sem):
            pltpu.make_async_copy(buf, agg_s.at[dst_v.at[0]], sem).wait()
            if with_cnt:
                pltpu.make_async_copy(ones_v, cnt_s.at[dst_v.at[0]],
                                      sem).wait()

        row0 = sid * PAIR + cid * W0
        nblk = jnp.where(cid == 0, W0 // BI, W1 // BI)

        def blk(b, carry):
            r = pl.multiple_of(row0 + b * BI, 8)
            pltpu.sync_copy(src_hbm.at[pl.ds(r, BI)], src_v)
            pltpu.sync_copy(dst_hbm.at[pl.ds(r, BI)], dst_v)
            bufs = [(bufA, gsA, ssA), (bufB, gsB, ssB)]
            gather(0, bufA, gsA)
            for j in range(BI):
                X, gX, sX = bufs[j % 2]
                Y, gY, sY = bufs[1 - j % 2]
                gwait(X, gX)
                if j < BI - 1:
                    if j >= 1:
                        swait(Y, sY)
                    gather(j + 1, Y, gY)
                scat(j, X, sX)
            swait(bufB, ssB)
            return carry

        lax.fori_loop(0, nblk, blk, 0)
        plsc.subcore_barrier()

        off = cid * NP + sid * SLAB
        pltpu.sync_copy(agg_s.at[pl.ds(sid * SLAB, SLAB)],
                        out_hbm.at[pl.ds(off, SLAB)])
        if with_cnt:
            pltpu.sync_copy(cnt_s.at[pl.ds(sid * SLAB, SLAB)],
                            cnt_hbm.at[pl.ds(off, SLAB)])

    return pl.kernel(body, out_type=out_type if with_cnt else out_type[0],
                     mesh=mesh, scratch_types=scratch)


_agg_cnt = _make_agg(True)
_agg = _make_agg(False)


def _tc0_body(p0, p1, c0, c1, h, wl, wr, b, out, cnt_out):
    cnt = jnp.maximum(c0[...] + c1[...], 1.0)
    mean = (p0[...] + p1[...]) / cnt
    acc = (jnp.dot(mean, wl[...], preferred_element_type=jnp.float32,
                   precision=jax.lax.Precision.HIGHEST)
           + jnp.dot(h[...], wr[...], preferred_element_type=jnp.float32,
                     precision=jax.lax.Precision.HIGHEST)
           + b[...])
    out[...] = jnp.maximum(acc, 0.0)
    cnt_out[...] = cnt


def _tc12_body(act, p0, p1, cnt, h, wl, wr, b, out):
    mean = (p0[...] + p1[...]) / cnt[...]
    acc = (jnp.dot(mean, wl[...], preferred_element_type=jnp.float32,
                   precision=jax.lax.Precision.HIGHEST)
           + jnp.dot(h[...], wr[...], preferred_element_type=jnp.float32,
                     precision=jax.lax.Precision.HIGHEST)
           + b[...])
    if act == "relu":
        out[...] = jnp.maximum(acc, 0.0)
    else:
        m = jnp.max(acc, axis=1, keepdims=True)
        s = acc - m
        out[...] = s - jnp.log(jnp.sum(jnp.exp(s), axis=1, keepdims=True))


_R = 1024
_GRID = (NP // _R,)
_rowD = pl.BlockSpec((_R, D), lambda i: (i, 0))
_row1 = pl.BlockSpec((_R, 1), lambda i: (i, 0))
_full = pl.BlockSpec((D, D), lambda i: (0, 0))
_bias = pl.BlockSpec((1, D), lambda i: (0, 0))


def _tc_layer0(p0, p1, c0, c1, h, wl, wr, b):
    return pl.pallas_call(
        _tc0_body,
        grid=_GRID,
        in_specs=[_rowD, _rowD, _row1, _row1, _rowD, _full, _full, _bias],
        out_specs=[_rowD, _row1],
        out_shape=[jax.ShapeDtypeStruct((NP, D), jnp.float32),
                   jax.ShapeDtypeStruct((NP, 1), jnp.float32)],
    )(p0, p1, c0, c1, h, wl, wr, b)


def _tc_layer(act, p0, p1, cnt, h, wl, wr, b):
    return pl.pallas_call(
        functools.partial(_tc12_body, act),
        grid=_GRID,
        in_specs=[_rowD, _rowD, _row1, _rowD, _full, _full, _bias],
        out_specs=_rowD,
        out_shape=jax.ShapeDtypeStruct((NP, D), jnp.float32),
    )(p0, p1, cnt, h, wl, wr, b)


def kernel(x, edge_index, W_l0, b_l0, W_r0, W_l1, b_l1, W_r1, W_l2, b_l2, W_r2):
    src = edge_index[0].astype(jnp.int32)
    dst = edge_index[1].astype(jnp.int32)
    npad_e = EPAD - E
    src = jnp.concatenate([src, jnp.zeros((npad_e,), jnp.int32)])
    dst = jnp.concatenate([dst, N + (jnp.arange(npad_e, dtype=jnp.int32) % (NP - N))])
    src2d = src.reshape(NROWS, CW)
    dst2d = dst.reshape(NROWS, CW)

    h0 = jnp.pad(x, ((0, NP - N), (0, 0)))
    znd = jnp.zeros((NP, D), jnp.float32)
    zn = jnp.zeros((NP,), jnp.float32)

    b0 = b_l0.reshape(1, D)
    b1 = b_l1.reshape(1, D)
    b2 = b_l2.reshape(1, D)

    p, c = _agg_cnt(h0, src2d, dst2d, znd, zn)
    h1, cnt = _tc_layer0(p[:NP], p[NP:], c[:NP, None], c[NP:, None],
                         h0, W_l0, W_r0, b0)
    p = _agg(h1, src2d, dst2d, znd)
    h2 = _tc_layer("relu", p[:NP], p[NP:], cnt, h1, W_l1, W_r1, b1)
    p = _agg(h2, src2d, dst2d, znd)
    h3 = _tc_layer("lsm", p[:NP], p[NP:], cnt, h2, W_l2, W_r2, b2)
    return h3[:N]


# speedup vs baseline: 1.1020x; 1.1020x over previous
import functools

import jax
import jax.numpy as jnp
from jax import lax
from jax.experimental import pallas as pl
from jax.experimental.pallas import tpu as pltpu
from jax.experimental.pallas import tpu_sc as plsc

N = 10000
E = 320000
D = 128

NC = 2
NS = 16
NW = NC * NS
NP = 10240
SLAB = NP // NS

CW = 64
PAIR = 320
W0 = 240
W1 = PAIR - W0
EPAD = NS * PAIR * CW
NROWS = EPAD // CW
BI = 16
NB = 4


def _make_agg(with_cnt: bool):
    mesh = plsc.VectorSubcoreMesh(core_axis_name="c", subcore_axis_name="s")

    out_type = [jax.ShapeDtypeStruct((NC * NP, D), jnp.float32)]
    scratch = [
        pltpu.VMEM((BI, CW), jnp.int32),
        pltpu.VMEM((BI, CW), jnp.int32),
        [pltpu.VMEM((CW, D), jnp.float32)] * NB,
        [pltpu.SemaphoreType.DMA] * NB,
        [pltpu.SemaphoreType.DMA] * NB,
        pltpu.VMEM_SHARED((NP, D), jnp.float32),
    ]
    if with_cnt:
        out_type.append(jax.ShapeDtypeStruct((NC * NP,), jnp.float32))
        scratch += [
            pltpu.VMEM((CW,), jnp.float32),
            pltpu.VMEM_SHARED((NP,), jnp.float32),
        ]

    def body(*refs):
        if with_cnt:
            (h_hbm, src_hbm, dst_hbm, znd_hbm, zn_hbm, out_hbm, cnt_hbm,
             src_v, dst_v, bufs, gsems, ssems, agg_s,
             ones_v, cnt_s) = refs
        else:
            (h_hbm, src_hbm, dst_hbm, znd_hbm, out_hbm,
             src_v, dst_v, bufs, gsems, ssems, agg_s) = refs

        cid = lax.axis_index("c")
        sid = lax.axis_index("s")
        wid = sid * NC + cid

        pltpu.sync_copy(znd_hbm.at[pl.ds(sid * SLAB, SLAB)],
                        agg_s.at[pl.ds(sid * SLAB, SLAB)])
        if with_cnt:
            pltpu.sync_copy(zn_hbm.at[pl.ds(sid * SLAB, SLAB)],
                            cnt_s.at[pl.ds(sid * SLAB, SLAB)])
            for i in range(CW // 16):
                ones_v[pl.ds(i * 16, 16)] = jnp.full((16,), 1.0, jnp.float32)
        plsc.subcore_barrier()

        def gather(j, buf, sem):
            pltpu.async_copy(h_hbm.at[src_v.at[j]], buf, sem)

        def gwait(buf, sem):
            pltpu.make_async_copy(h_hbm.at[src_v.at[0]], buf, sem).wait()

        def scat(j, buf, sem):
            pltpu.async_copy(buf, agg_s.at[dst_v.at[j]], sem, add=True)
            if with_cnt:
                pltpu.async_copy(ones_v, cnt_s.at[dst_v.at[j]], sem, add=True)

        def swait(buf, sem):
            pltpu.make_async_copy(buf, agg_s.at[dst_v.at[0]], sem).wait()
            if with_cnt:
                pltpu.make_async_copy(ones_v, cnt_s.at[dst_v.at[0]],
                                      sem).wait()

        row0 = sid * PAIR + cid * W0
        nblk = jnp.where(cid == 0, W0 // BI, W1 // BI)

        def blk(b, carry):
            r = pl.multiple_of(row0 + b * BI, 8)
            pltpu.sync_copy(src_hbm.at[pl.ds(r, BI)], src_v)
            pltpu.sync_copy(dst_hbm.at[pl.ds(r, BI)], dst_v)
            for j in range(NB - 1):
                gather(j, bufs[j], gsems[j])
            for j in range(BI):
                k = j % NB
                gwait(bufs[k], gsems[k])
                if j + NB - 1 < BI:
                    kn = (j + NB - 1) % NB
                    if j >= 1:
                        swait(bufs[kn], ssems[kn])
                    gather(j + NB - 1, bufs[kn], gsems[kn])
                scat(j, bufs[k], ssems[k])
            for j in range(BI - NB, BI):
                k = j % NB
                swait(bufs[k], ssems[k])
            return carry

        lax.fori_loop(0, nblk, blk, 0)
        plsc.subcore_barrier()

        off = cid * NP + sid * SLAB
        pltpu.sync_copy(agg_s.at[pl.ds(sid * SLAB, SLAB)],
                        out_hbm.at[pl.ds(off, SLAB)])
        if with_cnt:
            pltpu.sync_copy(cnt_s.at[pl.ds(sid * SLAB, SLAB)],
                            cnt_hbm.at[pl.ds(off, SLAB)])

    return pl.kernel(body, out_type=out_type if with_cnt else out_type[0],
                     mesh=mesh, scratch_types=scratch)


_agg_cnt = _make_agg(True)
_agg = _make_agg(False)


def _tc0_body(p0, p1, c0, c1, h, wl, wr, b, out, cnt_out):
    cnt = jnp.maximum(c0[...] + c1[...], 1.0)
    mean = (p0[...] + p1[...]) / cnt
    acc = (jnp.dot(mean, wl[...], preferred_element_type=jnp.float32,
                   precision=jax.lax.Precision.HIGHEST)
           + jnp.dot(h[...], wr[...], preferred_element_type=jnp.float32,
                     precision=jax.lax.Precision.HIGHEST)
           + b[...])
    out[...] = jnp.maximum(acc, 0.0)
    cnt_out[...] = cnt


def _tc12_body(act, p0, p1, cnt, h, wl, wr, b, out):
    mean = (p0[...] + p1[...]) / cnt[...]
    acc = (jnp.dot(mean, wl[...], preferred_element_type=jnp.float32,
                   precision=jax.lax.Precision.HIGHEST)
           + jnp.dot(h[...], wr[...], preferred_element_type=jnp.float32,
                     precision=jax.lax.Precision.HIGHEST)
           + b[...])
    if act == "relu":
        out[...] = jnp.maximum(acc, 0.0)
    else:
        m = jnp.max(acc, axis=1, keepdims=True)
        s = acc - m
        out[...] = s - jnp.log(jnp.sum(jnp.exp(s), axis=1, keepdims=True))


_R = 1024
_GRID = (NP // _R,)
_rowD = pl.BlockSpec((_R, D), lambda i: (i, 0))
_row1 = pl.BlockSpec((_R, 1), lambda i: (i, 0))
_full = pl.BlockSpec((D, D), lambda i: (0, 0))
_bias = pl.BlockSpec((1, D), lambda i: (0, 0))


def _tc_layer0(p0, p1, c0, c1, h, wl, wr, b):
    return pl.pallas_call(
        _tc0_body,
        grid=_GRID,
        in_specs=[_rowD, _rowD, _row1, _row1, _rowD, _full, _full, _bias],
        out_specs=[_rowD, _row1],
        out_shape=[jax.ShapeDtypeStruct((NP, D), jnp.float32),
                   jax.ShapeDtypeStruct((NP, 1), jnp.float32)],
    )(p0, p1, c0, c1, h, wl, wr, b)


def _tc_layer(act, p0, p1, cnt, h, wl, wr, b):
    return pl.pallas_call(
        functools.partial(_tc12_body, act),
        grid=_GRID,
        in_specs=[_rowD, _rowD, _row1, _rowD, _full, _full, _bias],
        out_specs=_rowD,
        out_shape=jax.ShapeDtypeStruct((NP, D), jnp.float32),
    )(p0, p1, cnt, h, wl, wr, b)


def kernel(x, edge_index, W_l0, b_l0, W_r0, W_l1, b_l1, W_r1, W_l2, b_l2, W_r2):
    src = edge_index[0].astype(jnp.int32)
    dst = edge_index[1].astype(jnp.int32)
    npad_e = EPAD - E
    src = jnp.concatenate([src, jnp.zeros((npad_e,), jnp.int32)])
    dst = jnp.concatenate([dst, N + (jnp.arange(npad_e, dtype=jnp.int32) % (NP - N))])
    src2d = src.reshape(NROWS, CW)
    dst2d = dst.reshape(NROWS, CW)

    h0 = jnp.pad(x, ((0, NP - N), (0, 0)))
    znd = jnp.zeros((NP, D), jnp.float32)
    zn = jnp.zeros((NP,), jnp.float32)

    b0 = b_l0.reshape(1, D)
    b1 = b_l1.reshape(1, D)
    b2 = b_l2.reshape(1, D)

    p, c = _agg_cnt(h0, src2d, dst2d, znd, zn)
    h1, cnt = _tc_layer0(p[:NP], p[NP:], c[:NP, None], c[NP:, None],
                         h0, W_l0, W_r0, b0)
    p = _agg(h1, src2d, dst2d, znd)
    h2 = _tc_layer("relu", p[:NP], p[NP:], cnt, h1, W_l1, W_r1, b1)
    p = _agg(h2, src2d, dst2d, znd)
    h3 = _tc_layer("lsm", p[:NP], p[NP:], cnt, h2, W_l2, W_r2, b2)
    return h3[:N]
